# Initial kernel scaffold; baseline (speedup 1.0000x reference)
import jax
import jax.numpy as jnp
from jax.experimental import pallas as pl


def kernel(x, mask, embedding_table, prompt_embed, response_embed, W_pred):
    raise NotImplementedError("write your pallas kernel here")


# speedup vs baseline: 1.1345x; 1.1345x over previous
import functools

import jax
import jax.numpy as jnp
from jax import lax
from jax.experimental import pallas as pl
from jax.experimental.pallas import tpu as pltpu
from jax.experimental.pallas import tpu_sc as plsc

B = 4
S = 2048
D = 768
N = B * S
NC, NS = 2, 16
NW = NC * NS
TPW = N // NW
CH = 64
NCH = TPW // CH
NJ = D // 16

_mesh = plsc.VectorSubcoreMesh(core_axis_name="c", subcore_axis_name="s")


@functools.partial(
    pl.kernel,
    mesh=_mesh,
    out_type=[
        jax.ShapeDtypeStruct((NW, 16), jnp.float32),
        jax.ShapeDtypeStruct((NW, 16), jnp.float32),
        jax.ShapeDtypeStruct((NW, 16), jnp.float32),
    ],
    scratch_types=[
        pltpu.VMEM((NCH, CH), jnp.int32),
        pltpu.VMEM((TPW,), jnp.int32),
        pltpu.VMEM((CH, D), jnp.float32),
        pltpu.VMEM((CH, D), jnp.float32),
        pltpu.VMEM((D,), jnp.float32),
        pltpu.VMEM((1, D), jnp.float32),
        pltpu.VMEM((16,), jnp.float32),
        pltpu.VMEM((16,), jnp.float32),
        pltpu.VMEM((16,), jnp.float32),
        pltpu.SemaphoreType.DMA,
        pltpu.SemaphoreType.DMA,
    ],
)
def _sc_pool(x_hbm, mask_hbm, table_hbm, w_hbm, p_hbm, d_hbm, e_hbm,
             idx_v, mask_v, rows0, rows1, w_v, e0_v, pout, dout, eout,
             gsem0, gsem1):
    wid = lax.axis_index("s") * NC + lax.axis_index("c")

    pltpu.sync_copy(x_hbm.at[wid], idx_v)
    pltpu.sync_copy(mask_hbm.at[wid], mask_v)
    pltpu.sync_copy(w_hbm, w_v)
    pltpu.sync_copy(table_hbm.at[pl.ds(0, 1)], e0_v)

    msum = jnp.zeros((16,), jnp.int32)
    for g in range(NCH):
        for t in range(CH // 16):
            m = mask_v[pl.ds(g * CH + t * 16, 16)]
            msum = msum + m
            sl = pl.ds(t * 16, 16)
            idx_v[g, sl] = idx_v[g, sl] * m

    rows = (rows0, rows1)
    gsems = (gsem0, gsem1)
    copies = [None, None]
    copies[0] = pltpu.async_copy(table_hbm.at[idx_v.at[0]], rows[0], gsems[0])

    accs = tuple(jnp.zeros((16,), jnp.float32) for _ in range(NJ))
    for g in range(NCH):
        if g + 1 < NCH:
            nb = (g + 1) % 2
            copies[nb] = pltpu.async_copy(
                table_hbm.at[idx_v.at[g + 1]], rows[nb], gsems[nb])
        copies[g % 2].wait()
        rbuf = rows[g % 2]

        def body(r, acc_t):
            return tuple(
                a + rbuf[r, pl.ds(j * 16, 16)] for j, a in enumerate(acc_t))

        accs = lax.fori_loop(0, CH, body, accs)

    dot = jnp.zeros((16,), jnp.float32)
    e0w = jnp.zeros((16,), jnp.float32)
    for j in range(NJ):
        wj = w_v[pl.ds(j * 16, 16)]
        dot = dot + accs[j] * wj
        e0w = e0w + e0_v[0, pl.ds(j * 16, 16)] * wj

    pout[...] = dot
    dout[...] = msum.astype(jnp.float32)
    eout[...] = e0w
    pltpu.sync_copy(pout, p_hbm.at[wid])
    pltpu.sync_copy(dout, d_hbm.at[wid])
    pltpu.sync_copy(eout, e_hbm.at[wid])


def _finish_body(p_ref, d_ref, e_ref, o_ref):
    num = jnp.sum(p_ref[...], axis=1)
    cnt = jnp.sum(d_ref[...], axis=1)
    e0w = jnp.sum(e_ref[...][0:1, 0:16])
    num = num - (S - cnt) * e0w
    o_ref[...] = num / jnp.clip(cnt, 1e-5, None)


def kernel(x, mask, embedding_table, prompt_embed, response_embed, W_pred):
    x_r = x.astype(jnp.int32).reshape(NW, NCH, CH)
    mask_r = mask.astype(jnp.int32).reshape(NW, TPW)
    p, d, e = _sc_pool(x_r, mask_r, embedding_table, W_pred)
    pred = pl.pallas_call(
        _finish_body,
        out_shape=jax.ShapeDtypeStruct((B,), jnp.float32),
    )(p.reshape(B, NW // B * 16), d.reshape(B, NW // B * 16), e)
    return pred
